# Initial kernel scaffold; baseline (speedup 1.0000x reference)
import jax
import jax.numpy as jnp
from jax.experimental import pallas as pl


def kernel(x, edge_index, W1l, W1r, b1, W2l, W2r, b2, fcW, fcb):
    raise NotImplementedError("write your pallas kernel here")


# speedup vs baseline: 11.3684x; 11.3684x over previous
import functools

import jax
import jax.numpy as jnp
from jax import lax
from jax.experimental import pallas as pl
from jax.experimental.pallas import tpu as pltpu
from jax.experimental.pallas import tpu_sc as plsc

N = 10000
D = 128
NC = 2
NS = 16
L = 16
NW = NC * NS
N_PAD = 10240
ZROWS = N_PAD // NS


def _fill_const(ref, rows, cols, val):
    v = jnp.full((L,), val, jnp.float32)
    for i in range(rows):
        for j in range(cols // L):
            ref[i, pl.ds(j * L, L)] = v


@functools.lru_cache(maxsize=None)
def _make_sc_agg(wr: int):
    mesh = plsc.VectorSubcoreMesh(core_axis_name="c", subcore_axis_name="s",
                                  num_cores=NC, num_subcores=NS)
    out_type = jax.ShapeDtypeStruct((NC, N_PAD, D), jnp.float32)
    scratch = [
        pltpu.VMEM((16, 128), jnp.int32),
        pltpu.VMEM((16, 128), jnp.int32),
        pltpu.VMEM((128, D), jnp.float32),
        pltpu.VMEM((128, D), jnp.float32),
        pltpu.VMEM((16, D), jnp.float32),
        pltpu.VMEM_SHARED((N_PAD, D), jnp.float32),
        pltpu.SemaphoreType.DMA,
        pltpu.SemaphoreType.DMA,
        pltpu.SemaphoreType.DMA,
        pltpu.SemaphoreType.DMA,
    ]

    def body(srcv, dstv, table, agg_out, sidx, didx, rows_a, rows_b, zbuf,
             agg_sh, sem_a, sem_b, ssem_a, ssem_b):
        c = lax.axis_index("c")
        s = lax.axis_index("s")
        wid = c * NS + s
        rows = (rows_a, rows_b)
        sems = (sem_a, sem_b)
        ssems = (ssem_a, ssem_b)

        _fill_const(zbuf, 16, D, 0.0)

        def zloop(i, _):
            pltpu.sync_copy(zbuf, agg_sh.at[pl.ds(s * ZROWS + i * 16, 16)])
            return 0
        lax.fori_loop(0, ZROWS // 16, zloop, 0)

        base = wid * wr
        plsc.subcore_barrier()

        nch = wr // 8

        def g_wait(b):
            pltpu.make_async_copy(table.at[sidx.at[0]], rows[b],
                                  sems[b]).wait()

        def s_wait(b):
            pltpu.make_async_copy(rows[b], agg_sh.at[didx.at[0]],
                                  ssems[b]).wait()

        pltpu.sync_copy(srcv.at[pl.ds(base, 8)], sidx.at[pl.ds(0, 8)])
        pltpu.sync_copy(dstv.at[pl.ds(base, 8)], didx.at[pl.ds(0, 8)])
        pltpu.async_copy(table.at[sidx.at[0]], rows[0], sems[0])

        def eloop(rr, _):
            po = (rr % 2) * 8
            no = 8 - po
            for k in range(8):
                b = k % 2
                g_wait(b)
                if k == 0:
                    @pl.when(rr > 0)
                    def _():
                        s_wait(1)
                else:
                    s_wait((k - 1) % 2)
                pltpu.async_copy(rows[b], agg_sh.at[didx.at[po + k]],
                                 ssems[b], add=True)
                if k < 7:
                    pltpu.async_copy(table.at[sidx.at[po + k + 1]],
                                     rows[(k + 1) % 2], sems[(k + 1) % 2])
                else:
                    @pl.when(rr + 1 < nch)
                    def _():
                        pltpu.async_copy(table.at[sidx.at[no]], rows[0],
                                         sems[0])
                if k == 0:
                    @pl.when(rr + 1 < nch)
                    def _():
                        nb = base + (rr + 1) * 8
                        pltpu.sync_copy(srcv.at[pl.ds(nb, 8)],
                                        sidx.at[pl.ds(no, 8)])
                        pltpu.sync_copy(dstv.at[pl.ds(nb, 8)],
                                        didx.at[pl.ds(no, 8)])
            return 0
        lax.fori_loop(0, nch, eloop, 0)
        s_wait(1)

        plsc.subcore_barrier()

        def wloop(i, _):
            off = s * ZROWS + i * 128
            pltpu.sync_copy(agg_sh.at[pl.ds(off, 128)],
                            agg_out.at[c, pl.ds(off, 128)])
            return 0
        lax.fori_loop(0, ZROWS // 128, wloop, 0)

    return pl.kernel(body, out_type=out_type, mesh=mesh,
                     scratch_types=scratch)


CROWS = N_PAD // 128


@functools.lru_cache(maxsize=None)
def _make_sc_cnt(wr: int):
    mesh = plsc.VectorSubcoreMesh(core_axis_name="c", subcore_axis_name="s",
                                  num_cores=NC, num_subcores=NS)
    out_type = jax.ShapeDtypeStruct((NC, N_PAD, L), jnp.float32)
    scratch = [
        pltpu.VMEM((8, 128), jnp.int32),
        pltpu.VMEM((CROWS, 128), jnp.float32),
        pltpu.VMEM((128, L), jnp.float32),
        pltpu.VMEM((CROWS,), jnp.int32),
        pltpu.VMEM_SHARED((CROWS, 128), jnp.float32),
    ]

    def body(dstv, cnt_out, didx, hist, wb, iidx, cnt_sh):
        c = lax.axis_index("c")
        s = lax.axis_index("s")
        wid = c * NS + s

        def hz(i, _):
            for j in range(8):
                hist[i, pl.ds(j * L, L)] = jnp.zeros((L,), jnp.float32)
            return 0
        lax.fori_loop(0, CROWS, hz, 0)

        @pl.when(s < CROWS // 8)
        def _():
            pltpu.sync_copy(hist.at[pl.ds(0, 8)],
                            cnt_sh.at[pl.ds(s * 8, 8)])

        base = wid * wr
        plsc.subcore_barrier()

        def eloop(rr, _):
            pltpu.sync_copy(dstv.at[pl.ds(base + rr * 8, 8)], didx)
            for k in range(8):
                for j in range(8):
                    d = didx[k, pl.ds(j * L, L)]
                    row = lax.shift_right_logical(d, 7)
                    lane = lax.bitwise_and(d, 127)
                    plsc.addupdate_scatter(hist, [row, lane],
                                           jnp.full((L,), 1.0, jnp.float32))
            return 0
        lax.fori_loop(0, wr // 8, eloop, 0)

        for j in range(CROWS // L):
            iidx[pl.ds(j * L, L)] = lax.iota(jnp.int32, L) + j * L
        pltpu.sync_copy(hist, cnt_sh.at[iidx], add=True)
        plsc.subcore_barrier()

        pltpu.sync_copy(cnt_sh, hist)

        def wloop(g, _):
            def nloop(i, _):
                rsp = jnp.full((L,), s * (ZROWS // 128) + g, jnp.int32)
                lsp = jnp.full((L,), i, jnp.int32)
                wb[i, :] = plsc.load_gather(hist, [rsp, lsp])
                return 0
            lax.fori_loop(0, 128, nloop, 0)
            pltpu.sync_copy(
                wb, cnt_out.at[c, pl.ds(s * ZROWS + g * 128, 128)])
            return 0
        lax.fori_loop(0, ZROWS // 128, wloop, 0)

    return pl.kernel(
        body, out_type=out_type, mesh=mesh, scratch_types=scratch,
        compiler_params=pltpu.CompilerParams(needs_layout_passes=False))


_B = 1000


def _tc1_body(agg_ref, cnt_ref, x_ref, wl_ref, wr_ref, b_ref, o_ref):
    agg = agg_ref[0] + agg_ref[1]
    cnt = cnt_ref[0, :, 0:1] + cnt_ref[1, :, 0:1]
    mean = agg / jnp.maximum(cnt, 1.0)
    h = (jnp.dot(mean, wl_ref[...], preferred_element_type=jnp.float32)
         + jnp.dot(x_ref[...], wr_ref[...], preferred_element_type=jnp.float32)
         + b_ref[...])
    o_ref[...] = jnp.maximum(h, 0.0)


def _tc2_body(agg_ref, cnt_ref, h_ref, wl_ref, wr_ref, b_ref, fw_ref, fb_ref,
              o_ref):
    agg = agg_ref[0] + agg_ref[1]
    cnt = cnt_ref[0, :, 0:1] + cnt_ref[1, :, 0:1]
    mean = agg / jnp.maximum(cnt, 1.0)
    g = (jnp.dot(mean, wl_ref[...], preferred_element_type=jnp.float32)
         + jnp.dot(h_ref[...], wr_ref[...], preferred_element_type=jnp.float32)
         + b_ref[...])
    o_ref[...] = (jnp.dot(g, fw_ref[...], preferred_element_type=jnp.float32)
                  + fb_ref[...])


def _w_spec():
    return pl.BlockSpec((D, D), lambda i: (0, 0))


def _b_spec():
    return pl.BlockSpec((1, D), lambda i: (0, 0))


@functools.lru_cache(maxsize=None)
def _make_tc1():
    return pl.pallas_call(
        _tc1_body,
        grid=(N // _B,),
        in_specs=[
            pl.BlockSpec((NC, _B, D), lambda i: (0, i, 0)),
            pl.BlockSpec((NC, _B, L), lambda i: (0, i, 0)),
            pl.BlockSpec((_B, D), lambda i: (i, 0)),
            _w_spec(), _w_spec(), _b_spec(),
        ],
        out_specs=pl.BlockSpec((_B, D), lambda i: (i, 0)),
        out_shape=jax.ShapeDtypeStruct((N, D), jnp.float32),
    )


@functools.lru_cache(maxsize=None)
def _make_tc2():
    return pl.pallas_call(
        _tc2_body,
        grid=(N // _B,),
        in_specs=[
            pl.BlockSpec((NC, _B, D), lambda i: (0, i, 0)),
            pl.BlockSpec((NC, _B, L), lambda i: (0, i, 0)),
            pl.BlockSpec((_B, D), lambda i: (i, 0)),
            _w_spec(), _w_spec(), _b_spec(), _w_spec(), _b_spec(),
        ],
        out_specs=pl.BlockSpec((_B, D), lambda i: (i, 0)),
        out_shape=jax.ShapeDtypeStruct((N, D), jnp.float32),
    )


def kernel(x, edge_index, W1l, W1r, b1, W2l, W2r, b2, fcW, fcb):
    src = edge_index[0].astype(jnp.int32)
    dst = edge_index[1].astype(jnp.int32)
    e = src.shape[0]
    wr = -(-e // (NW * 128 * 8)) * 8
    ep = NW * 128 * wr
    pad = ep - e
    pad_i = jnp.arange(pad, dtype=jnp.int32)
    srcp = jnp.concatenate([src, pad_i % N]).reshape(NW * wr, 128)
    dstp = jnp.concatenate([dst, N + pad_i % (N_PAD - N)]).reshape(NW * wr, 128)

    cnt = _make_sc_cnt(wr)(dstp)
    agg1 = _make_sc_agg(wr)(srcp, dstp, x)
    h = _make_tc1()(agg1, cnt, x, W1l, W1r, b1.reshape(1, D))
    agg2 = _make_sc_agg(wr)(srcp, dstp, h)
    return _make_tc2()(agg2, cnt, h, W2l, W2r, b2.reshape(1, D),
                       fcW, fcb.reshape(1, D))
